# baseline (speedup 1.0000x reference)
import functools

import jax
import jax.numpy as jnp
from jax import lax
from jax.experimental import pallas as pl
from jax.experimental.pallas import tpu as pltpu

N = 10000
E = 320000
F_IN = 128
HID = 128
NCLS = 16
K = 3
NGRAPH = 64

_MCHUNK = 6400


def _cheb_conv(x, src, dst, W, b, mask=None):
    n = x.shape[0]
    ones = jnp.ones(src.shape[0], jnp.float32) if mask is None else mask
    deg = jax.ops.segment_sum(ones, dst, num_segments=n)
    dis = jnp.where(deg > 0, 1.0 / jnp.sqrt(jnp.maximum(deg, 1.0)), 0.0)
    w = -dis[src] * dis[dst]
    if mask is not None:
        w = w * mask

    def lap(h):
        return jax.ops.segment_sum(w[:, None] * h[src], dst, num_segments=n)

    Tx0 = x
    out = Tx0 @ W[0]
    Tx1 = lap(Tx0)
    out = out + Tx1 @ W[1]
    Tx2 = 2.0 * lap(Tx1) - Tx0
    out = out + Tx2 @ W[2]
    return out + b


def _match_body(edges_ref, m_ref):
    step = pl.program_id(0)

    @pl.when(step == 0)
    def _init():
        def initb(i, _):
            m_ref[i] = -1
            return 0

        lax.fori_loop(0, N, initb, 0, unroll=8)

    def body(i, _):
        u = edges_ref[0, i]
        v = edges_ref[1, i]
        mu = m_ref[u]
        mv = m_ref[v]
        ok = (u != v) & (mu < 0) & (mv < 0)
        m_ref[u] = jnp.where(ok, v, mu)
        m_ref[v] = jnp.where(ok, u, mv)
        return 0

    lax.fori_loop(0, _MCHUNK, body, 0)


def _graclus_pallas(edge_index):
    m = pl.pallas_call(
        _match_body,
        grid=(E // _MCHUNK,),
        in_specs=[
            pl.BlockSpec((2, _MCHUNK), lambda i: (0, i), memory_space=pltpu.SMEM),
        ],
        out_specs=pl.BlockSpec(memory_space=pltpu.SMEM),
        out_shape=jax.ShapeDtypeStruct((N,), jnp.int32),
    )(edge_index)
    idx = jnp.arange(N, dtype=jnp.int32)
    match = jnp.where(m < 0, idx, m)
    rep = jnp.minimum(idx, match)
    return rep


def _final_kernel(pooled_ref, wfc_ref, bfc_ref, out_ref):
    logits = pooled_ref[...] @ wfc_ref[...] + bfc_ref[...][None, :]
    mx = jnp.max(logits, axis=1, keepdims=True)
    s = logits - mx
    lse = jnp.log(jnp.sum(jnp.exp(s), axis=1, keepdims=True))
    out_ref[...] = s - lse


def kernel(x, edge_index, batch, w1, b1, w2, b2, wfc, bfc):
    n = x.shape[0]
    src = edge_index[0]
    dst = edge_index[1]
    h = jax.nn.relu(_cheb_conv(x, src, dst, w1, b1))
    rep = _graclus_pallas(edge_index)
    counts = jax.ops.segment_sum(jnp.ones(n, jnp.float32), rep, num_segments=n)
    hc = jax.ops.segment_sum(h, rep, num_segments=n) / jnp.maximum(counts, 1.0)[:, None]
    ns = rep[src]
    nd = rep[dst]
    emask = (ns != nd).astype(jnp.float32)
    h2 = jax.nn.relu(_cheb_conv(hc, ns, nd, w2, b2, emask))
    valid = (rep == jnp.arange(n, dtype=jnp.int32)).astype(jnp.float32)
    gcnt = jax.ops.segment_sum(valid, batch, num_segments=NGRAPH)
    pooled = jax.ops.segment_sum(h2, batch, num_segments=NGRAPH) / jnp.maximum(gcnt, 1.0)[:, None]
    out = pl.pallas_call(
        _final_kernel,
        out_shape=jax.ShapeDtypeStruct((NGRAPH, NCLS), jnp.float32),
    )(pooled, wfc, bfc)
    return out


# speedup vs baseline: 3.7223x; 3.7223x over previous
import functools

import jax
import jax.numpy as jnp
from jax import lax
from jax.experimental import pallas as pl
from jax.experimental.pallas import tpu as pltpu
from jax.experimental.pallas import tpu_sc as plsc

N = 10000
E = 320000
F_IN = 128
HID = 128
NCLS = 16
NGRAPH = 64

NW = 32
CH = 128
_MCHUNK = 6400

ROWS_PAD = 10240
DUMMY = N
EPAD_N = 10240
MP_NODE = 12288
GROWS_PAD = 128
MP_G = 2048


def _mesh():
    return plsc.VectorSubcoreMesh(core_axis_name="c", subcore_axis_name="s")


def _zero_fill(ref, nrows):
    zv = jnp.zeros((16,), jnp.float32)

    def body(i, _):
        r = i // 8
        col = (i % 8) * 16
        ref[r, pl.ds(col, 16)] = zv
        return 0

    lax.fori_loop(0, nrows * 8, body, 0)


def _zero_fill_1d(ref, nwords):
    zv = jnp.zeros((16,), jnp.float32)

    def body(i, _):
        ref[pl.ds(i * 16, 16)] = zv
        return 0

    lax.fori_loop(0, nwords // 16, body, 0)


def _make_segsum_rows(rows_pad, e_pad):
    nchunk_total = e_pad // CH
    zr = rows_pad // 16
    zstep = min(zr, 64)
    nsc = 16

    @functools.partial(
        pl.kernel,
        mesh=plsc.VectorSubcoreMesh(
            core_axis_name="c", subcore_axis_name="s", num_cores=1
        ),
        out_type=jax.ShapeDtypeStruct((rows_pad, 128), jnp.float32),
        scratch_types=[
            pltpu.VMEM((CH,), jnp.int32),
            pltpu.VMEM((CH,), jnp.int32),
            pltpu.VMEM((CH, 128), jnp.float32),
            pltpu.VMEM((64, 128), jnp.float32),
            pltpu.VMEM_SHARED((rows_pad, 128), jnp.float32),
        ],
    )
    def k(table, srcidx, dstidx, out, sbuf, dbuf, rows, zbuf, acc):
        s = lax.axis_index("s")
        _zero_fill(zbuf, zstep)

        def zcp(t, _):
            pltpu.sync_copy(
                zbuf.at[pl.ds(0, zstep)], acc.at[pl.ds(s * zr + t * zstep, zstep)]
            )
            return 0

        lax.fori_loop(0, zr // zstep, zcp, 0)
        plsc.subcore_barrier()

        def chunk(i, _):
            base = (s + i * nsc) * CH
            pltpu.sync_copy(srcidx.at[pl.ds(base, CH)], sbuf)
            pltpu.sync_copy(dstidx.at[pl.ds(base, CH)], dbuf)
            pltpu.sync_copy(table.at[sbuf], rows)
            pltpu.sync_copy(rows, acc.at[dbuf], add=True)
            return 0

        nchunk = (nchunk_total - s + nsc - 1) // nsc
        lax.fori_loop(0, nchunk, chunk, 0)
        plsc.subcore_barrier()
        pltpu.sync_copy(acc.at[pl.ds(s * zr, zr)], out.at[pl.ds(s * zr, zr)])

    return k


def _make_segsum_scalar(mp, e_pad):
    nchunk_total = e_pad // CH
    zw = mp // 16

    @functools.partial(
        pl.kernel,
        mesh=_mesh(),
        out_type=jax.ShapeDtypeStruct((2, mp), jnp.float32),
        scratch_types=[
            pltpu.VMEM((CH,), jnp.int32),
            pltpu.VMEM((CH,), jnp.float32),
            pltpu.VMEM((zw,), jnp.float32),
            pltpu.VMEM_SHARED((mp,), jnp.float32),
        ],
    )
    def k(vals, idx, out, ibuf, vbuf, zbuf, acc):
        c = lax.axis_index("c")
        s = lax.axis_index("s")
        wid = c * 16 + s
        _zero_fill_1d(zbuf, zw)
        pltpu.sync_copy(zbuf, acc.at[pl.ds(s * zw, zw)])
        plsc.subcore_barrier()

        def chunk(i, _):
            base = (wid + i * NW) * CH
            pltpu.sync_copy(idx.at[pl.ds(base, CH)], ibuf)
            pltpu.sync_copy(vals.at[pl.ds(base, CH)], vbuf)
            pltpu.sync_copy(vbuf, acc.at[ibuf], add=True)
            return 0

        nchunk = (nchunk_total - wid + NW - 1) // NW
        lax.fori_loop(0, nchunk, chunk, 0)
        plsc.subcore_barrier()
        pltpu.sync_copy(acc.at[pl.ds(s * zw, zw)], out.at[c].at[pl.ds(s * zw, zw)])

    return k


def _make_relabel():
    nchunk_total = E // CH
    zw = MP_NODE // 16

    @functools.partial(
        pl.kernel,
        mesh=_mesh(),
        out_type=(
            jax.ShapeDtypeStruct((E,), jnp.int32),
            jax.ShapeDtypeStruct((E,), jnp.int32),
            jax.ShapeDtypeStruct((2, MP_NODE), jnp.float32),
        ),
        scratch_types=[
            pltpu.VMEM((N,), jnp.int32),
            pltpu.VMEM((CH,), jnp.int32),
            pltpu.VMEM((CH,), jnp.int32),
            pltpu.VMEM((CH,), jnp.int32),
            pltpu.VMEM((CH,), jnp.int32),
            pltpu.VMEM((CH,), jnp.float32),
            pltpu.VMEM((zw,), jnp.float32),
            pltpu.VMEM_SHARED((MP_NODE,), jnp.float32),
        ],
    )
    def k(rep, srcidx, dstidx, ns_out, nd_out, deg_out,
          repbuf, sbuf, dbuf, nsbuf, ndbuf, onebuf, zbuf, acc):
        c = lax.axis_index("c")
        s = lax.axis_index("s")
        wid = c * 16 + s
        _zero_fill_1d(zbuf, zw)
        pltpu.sync_copy(zbuf, acc.at[pl.ds(s * zw, zw)])
        pltpu.sync_copy(rep, repbuf)
        plsc.subcore_barrier()

        def chunk(i, _):
            base = (wid + i * NW) * CH
            pltpu.sync_copy(srcidx.at[pl.ds(base, CH)], sbuf)
            pltpu.sync_copy(dstidx.at[pl.ds(base, CH)], dbuf)

            def vec(j, _2):
                su = sbuf[pl.ds(j * 16, 16)]
                dv = dbuf[pl.ds(j * 16, 16)]
                ns = plsc.load_gather(repbuf, [su])
                nd = plsc.load_gather(repbuf, [dv])
                keep = ns != nd
                ndeff = jnp.where(keep, nd, DUMMY)
                one = jnp.where(keep, 1.0, 0.0)
                nsbuf[pl.ds(j * 16, 16)] = ns
                ndbuf[pl.ds(j * 16, 16)] = ndeff
                onebuf[pl.ds(j * 16, 16)] = one
                return 0

            lax.fori_loop(0, CH // 16, vec, 0)
            pltpu.sync_copy(nsbuf, ns_out.at[pl.ds(base, CH)])
            pltpu.sync_copy(ndbuf, nd_out.at[pl.ds(base, CH)])
            pltpu.sync_copy(onebuf, acc.at[ndbuf], add=True)
            return 0

        nchunk = (nchunk_total - wid + NW - 1) // NW
        lax.fori_loop(0, nchunk, chunk, 0)
        plsc.subcore_barrier()
        pltpu.sync_copy(acc.at[pl.ds(s * zw, zw)], deg_out.at[c].at[pl.ds(s * zw, zw)])

    return k


_segsum_lap = _make_segsum_rows(ROWS_PAD, E)
_segsum_pool = _make_segsum_rows(ROWS_PAD, EPAD_N)
_segsum_graph = _make_segsum_rows(GROWS_PAD, EPAD_N)
_segsum_deg = _make_segsum_scalar(MP_NODE, E)
_segsum_cnt = _make_segsum_scalar(MP_NODE, EPAD_N)
_segsum_gcnt = _make_segsum_scalar(MP_G, EPAD_N)
_relabel = _make_relabel()


def _match_body(edges_ref, m_ref):
    step = pl.program_id(0)

    @pl.when(step == 0)
    def _init():
        def initb(i, _):
            m_ref[i] = -1
            return 0

        lax.fori_loop(0, N, initb, 0, unroll=8)

    def body(i, _):
        u = edges_ref[0, i]
        v = edges_ref[1, i]
        mu = m_ref[u]
        mv = m_ref[v]
        ok = (u != v) & (mu < 0) & (mv < 0)
        m_ref[u] = jnp.where(ok, v, mu)
        m_ref[v] = jnp.where(ok, u, mv)
        return 0

    lax.fori_loop(0, _MCHUNK, body, 0)


def _graclus_pallas(edge_index):
    return pl.pallas_call(
        _match_body,
        grid=(E // _MCHUNK,),
        in_specs=[
            pl.BlockSpec((2, _MCHUNK), lambda i: (0, i), memory_space=pltpu.SMEM),
        ],
        out_specs=pl.BlockSpec(memory_space=pltpu.SMEM),
        out_shape=jax.ShapeDtypeStruct((N,), jnp.int32),
    )(edge_index)


_RB = 400


def _tc_call(body, nout, extra_full_inputs, blocked_inputs, out_shapes):
    grid = (N // _RB,)
    in_specs = []
    for shp in blocked_inputs:
        in_specs.append(
            pl.BlockSpec(
                (_RB,) + shp[1:], lambda i, n=len(shp) - 1: (i,) + (0,) * n
            )
        )
    for shp in extra_full_inputs:
        in_specs.append(pl.BlockSpec(shp, lambda i, n=len(shp): (0,) * n))
    out_specs = [
        pl.BlockSpec((_RB,) + shp[1:], lambda i, n=len(shp) - 1: (i,) + (0,) * n)
        for shp in out_shapes
    ]
    return pl.pallas_call(
        body,
        grid=grid,
        in_specs=in_specs,
        out_specs=out_specs if nout > 1 else out_specs[0],
        out_shape=[jax.ShapeDtypeStruct(s, jnp.float32) for s in out_shapes]
        if nout > 1
        else jax.ShapeDtypeStruct(out_shapes[0], jnp.float32),
    )


def _dis_from_deg(deg):
    return jnp.where(deg > 0, lax.rsqrt(jnp.maximum(deg, 1.0)), 0.0)


def _stage1_body(deg0_ref, deg1_ref, x_ref, w0_ref, dis_ref, g_ref, out0_ref):
    deg = deg0_ref[...] + deg1_ref[...]
    dis = _dis_from_deg(deg)
    dis_ref[...] = dis
    g_ref[...] = x_ref[...] * dis
    out0_ref[...] = jnp.dot(x_ref[...], w0_ref[...], preferred_element_type=jnp.float32)


def _stage2_body(s_ref, dis_ref, out0_ref, w1_ref, g1_ref, acc_ref, tx1_ref):
    dis = dis_ref[...]
    tx1 = -dis * s_ref[...]
    g1_ref[...] = dis * tx1
    tx1_ref[...] = tx1
    acc_ref[...] = out0_ref[...] + jnp.dot(tx1, w1_ref[...], preferred_element_type=jnp.float32)


def _stage3_body(s_ref, dis_ref, x_ref, acc_ref, w2_ref, b_ref, h_ref):
    dis = dis_ref[...]
    tx2 = -2.0 * dis * s_ref[...] - x_ref[...]
    pre = acc_ref[...] + jnp.dot(tx2, w2_ref[...], preferred_element_type=jnp.float32) + b_ref[...]
    h_ref[...] = jnp.maximum(pre, 0.0)


def _pool_body(hs_ref, cnt0_ref, cnt1_ref, rep_ref, iota_ref, hc_ref, valid_ref):
    cnt = cnt0_ref[...] + cnt1_ref[...]
    hc_ref[...] = hs_ref[...] / jnp.maximum(cnt, 1.0)
    valid_ref[...] = jnp.where(rep_ref[...] == iota_ref[...], 1.0, 0.0)


def _final_body(p_ref, g0_ref, g1_ref, wfc_ref, bfc_ref, out_ref):
    gcnt = g0_ref[...] + g1_ref[...]
    pooled = p_ref[...] / jnp.maximum(gcnt, 1.0)
    logits = jnp.dot(pooled, wfc_ref[...], preferred_element_type=jnp.float32) + bfc_ref[...]
    mx = jnp.max(logits, axis=1, keepdims=True)
    sh = logits - mx
    lse = jnp.log(jnp.sum(jnp.exp(sh), axis=1, keepdims=True))
    out_ref[...] = sh - lse


def _cheb_layer(x, g_table_src, src, dst, deg_parts, W, b):
    dis, g0, out0 = _tc_call(
        _stage1_body, 3,
        extra_full_inputs=[(128, 128)],
        blocked_inputs=[(N, 1), (N, 1), (N, 128)],
        out_shapes=[(N, 1), (N, 128), (N, 128)],
    )(deg_parts[0], deg_parts[1], x, W[0])

    s1 = g_table_src(g0, src, dst)
    g1, acc, _tx1 = _tc_call(
        _stage2_body, 3,
        extra_full_inputs=[(128, 128)],
        blocked_inputs=[(N, 128), (N, 1), (N, 128)],
        out_shapes=[(N, 128), (N, 128), (N, 128)],
    )(s1, dis, out0, W[1])

    s2 = g_table_src(g1, src, dst)
    h = _tc_call(
        _stage3_body, 1,
        extra_full_inputs=[(128, 128), (1, 128)],
        blocked_inputs=[(N, 128), (N, 1), (N, 128), (N, 128)],
        out_shapes=[(N, 128)],
    )(s2, dis, x, acc, W[2], b.reshape(1, 128))
    return h


def _make_gather_words(e_pad):
    nchunk_total = e_pad // CH

    @functools.partial(
        pl.kernel,
        mesh=_mesh(),
        out_type=jax.ShapeDtypeStruct((e_pad,), jnp.int32),
        scratch_types=[
            pltpu.VMEM((CH,), jnp.int32),
            pltpu.VMEM((CH,), jnp.int32),
        ],
    )
    def k(table, idx, out, ibuf, obuf):
        c = lax.axis_index("c")
        s = lax.axis_index("s")
        wid = c * 16 + s

        def chunk(i, _):
            base = (wid + i * NW) * CH
            pltpu.sync_copy(idx.at[pl.ds(base, CH)], ibuf)
            pltpu.sync_copy(table.at[ibuf], obuf)
            pltpu.sync_copy(obuf, out.at[pl.ds(base, CH)])
            return 0

        nchunk = (nchunk_total - wid + NW - 1) // NW
        lax.fori_loop(0, nchunk, chunk, 0)

    return k


_gather_words = _make_gather_words(E)
_ER = 2500


def _mask_body(ns_ref, nd_ref, ndeff_ref, emask_ref):
    ns = ns_ref[...]
    nd = nd_ref[...]
    keep = ns != nd
    ndeff_ref[...] = jnp.where(keep, nd, DUMMY)
    emask_ref[...] = jnp.where(keep, 1.0, 0.0)


def _edge_mask(ns, nd):
    ns2 = ns.reshape(_ER, 128)
    nd2 = nd.reshape(_ER, 128)
    ndeff, emask = pl.pallas_call(
        _mask_body,
        out_shape=[
            jax.ShapeDtypeStruct((_ER, 128), jnp.int32),
            jax.ShapeDtypeStruct((_ER, 128), jnp.float32),
        ],
    )(ns2, nd2)
    return ndeff.reshape(E), emask.reshape(E)


def kernel(x, edge_index, batch, w1, b1, w2, b2, wfc, bfc):
    src = edge_index[0]
    dst = edge_index[1]
    ones_e = jnp.ones((E,), jnp.float32)

    deg1 = _segsum_deg(ones_e, dst)
    deg1p = (deg1[0, :N].reshape(N, 1), deg1[1, :N].reshape(N, 1))

    def lap1(g, sidx, didx):
        return _segsum_lap(g, sidx, didx)[:N, :]

    h = _cheb_layer(x, lap1, src, dst, deg1p, w1, b1)

    m = _graclus_pallas(edge_index)
    idx = jnp.arange(N, dtype=jnp.int32)
    match = jnp.where(m < 0, idx, m)
    rep = jnp.minimum(idx, match)

    rep_pad = jnp.concatenate([rep, jnp.full((EPAD_N - N,), DUMMY, jnp.int32)])
    iota_pad = jnp.concatenate([idx, jnp.zeros((EPAD_N - N,), jnp.int32)])
    ones_n = jnp.ones((EPAD_N,), jnp.float32)

    counts = _segsum_cnt(ones_n, rep_pad)
    hsum = _segsum_pool(h, iota_pad, rep_pad)[:N, :]
    hc, valid = _tc_call(
        _pool_body, 2,
        extra_full_inputs=[],
        blocked_inputs=[(N, 128), (N, 1), (N, 1), (N, 1), (N, 1)],
        out_shapes=[(N, 128), (N, 1)],
    )(
        hsum,
        counts[0, :N].reshape(N, 1), counts[1, :N].reshape(N, 1),
        rep.reshape(N, 1).astype(jnp.float32), idx.reshape(N, 1).astype(jnp.float32),
    )

    ns = _gather_words(rep, src)
    nd2c = _gather_words(rep, dst)
    ndeff, emask = _edge_mask(ns, nd2c)
    deg2 = _segsum_deg(emask, nd2c)
    deg2p = (deg2[0, :N].reshape(N, 1), deg2[1, :N].reshape(N, 1))

    h2 = _cheb_layer(hc, lap1, ns, ndeff, deg2p, w2, b2)

    batch_pad = jnp.concatenate(
        [batch, jnp.full((EPAD_N - N,), NGRAPH, jnp.int32)]
    )
    valid_pad = jnp.concatenate([valid.reshape(N), jnp.zeros((EPAD_N - N,), jnp.float32)])
    gcnt = _segsum_gcnt(valid_pad, batch_pad)
    psum = _segsum_graph(h2, iota_pad, batch_pad)[:NGRAPH, :]

    out = pl.pallas_call(
        _final_body,
        out_shape=jax.ShapeDtypeStruct((NGRAPH, NCLS), jnp.float32),
    )(
        psum,
        gcnt[0, :NGRAPH].reshape(NGRAPH, 1), gcnt[1, :NGRAPH].reshape(NGRAPH, 1),
        wfc, bfc.reshape(1, NCLS),
    )
    return out
